# baseline (speedup 1.0000x reference)
import functools

import jax
import jax.numpy as jnp
import numpy as np
from jax.experimental import pallas as pl

_NUM_BINS = 12
_ANGLE_PER_BIN = 2.0 * np.pi / _NUM_BINS
_TWO_PI = 2.0 * np.pi


def _rpn_kernel(x_ref, w_ref, out_ref):
    x = x_ref[...]
    w1 = w_ref[0:256, :]
    w2 = w_ref[256:384, :]
    wht = w_ref[384:415, :]
    bt = w_ref[416:419, :].T
    b1 = bt[:, 0:1]
    b2 = bt[:, 1:2]
    bh = bt[0:31, 2:3]

    h = jnp.maximum(
        jax.lax.dot_general(w1, x, (((0,), (1,)), ((), ())),
                            preferred_element_type=jnp.float32)
        + b1, 0.0)
    h = jnp.maximum(
        jax.lax.dot_general(w2, h, (((0,), (0,)), ((), ())),
                            preferred_element_type=jnp.float32)
        + b2, 0.0)
    o = (jax.lax.dot_general(wht, h, (((1,), (0,)), ((), ())),
                             preferred_element_type=jnp.float32)
         + bh)

    obj = jax.nn.sigmoid(o[0:1, :])

    hcls = o[7:7 + _NUM_BINS, :]
    hd = o[7 + _NUM_BINS:7 + 2 * _NUM_BINS, :]
    mx = jnp.max(hcls, axis=0, keepdims=True)
    iota = jax.lax.broadcasted_iota(jnp.int32, hcls.shape, 0)
    idx = jnp.min(jnp.where(hcls == mx, iota, _NUM_BINS),
                  axis=0, keepdims=True)
    delta = jnp.sum(jnp.where(iota == idx, hd, 0.0), axis=0, keepdims=True)
    ang = jnp.mod(idx.astype(jnp.float32) * _ANGLE_PER_BIN + delta, _TWO_PI)

    out_ref[...] = jnp.concatenate([obj, ang, o[1:7, :]], axis=0)


@functools.partial(jax.jit, static_argnames=())
def kernel(voted_xyz, voted_features, W1, b1, W2, b2, W_obj, b_obj,
           W_box, b_box, W_hcls, b_hcls, W_hd, b_hd):
    B, N, C = voted_features.shape
    H = W1.shape[1]
    M = B * N
    T = 4096
    grid = (M // T,)

    x = voted_features.reshape(M, C)

    f32 = W1.dtype
    bias_row = jnp.concatenate(
        [b_obj, b_box, b_hcls, b_hd, jnp.zeros((H - 31,), f32)])[None, :]
    packed = jnp.concatenate(
        [W1, W2, W_obj.T, W_box.T, W_hcls.T, W_hd.T,
         jnp.zeros((1, H), f32), b1[None, :], b2[None, :], bias_row,
         jnp.zeros((5, H), f32)], axis=0)

    out = pl.pallas_call(
        _rpn_kernel,
        grid=grid,
        in_specs=[
            pl.BlockSpec((T, C), lambda i: (i, 0)),
            pl.BlockSpec((424, H), lambda i: (0, 0)),
        ],
        out_specs=pl.BlockSpec((8, T), lambda i: (0, i)),
        out_shape=jax.ShapeDtypeStruct((8, M), jnp.float32),
    )(x, packed)

    obj = out[0].reshape(B, N)
    ang = out[1].reshape(B, N)
    d = out[2:8].T
    xyz = voted_xyz.reshape(M, 3)
    boxes = jnp.concatenate([xyz - d[:, 0:3], xyz + d[:, 3:6]],
                            axis=-1).reshape(B, N, 6)
    return (obj, boxes, ang)


# speedup vs baseline: 1.4520x; 1.4520x over previous
import functools

import jax
import jax.numpy as jnp
import numpy as np
from jax.experimental import pallas as pl

_NUM_BINS = 12
_ANGLE_PER_BIN = 2.0 * np.pi / _NUM_BINS
_TWO_PI = 2.0 * np.pi


def _rpn_kernel(x_ref, core_ref, heads_ref, out_ref):
    x = x_ref[...]
    w1 = core_ref[0:256, :]
    w2 = core_ref[256:384, :]
    bt = core_ref[384:392, :].T
    b1 = bt[:, 0:1]
    b2 = bt[:, 1:2]
    bh = bt[0:31, 2:3]

    h = jnp.maximum(
        jax.lax.dot_general(w1, x, (((0,), (1,)), ((), ())),
                            preferred_element_type=jnp.float32)
        + b1, 0.0)
    h = jnp.maximum(
        jax.lax.dot_general(w2, h, (((0,), (0,)), ((), ())),
                            preferred_element_type=jnp.float32)
        + b2, 0.0)
    o = (jax.lax.dot_general(heads_ref[...], h, (((0,), (0,)), ((), ())),
                             preferred_element_type=jnp.float32)
         + bh)

    obj = jax.nn.sigmoid(o[0:1, :])

    hcls = o[7:7 + _NUM_BINS, :]
    hd = o[7 + _NUM_BINS:7 + 2 * _NUM_BINS, :]
    mx = jnp.max(hcls, axis=0, keepdims=True)
    iota = jax.lax.broadcasted_iota(jnp.int32, hcls.shape, 0)
    idx = jnp.min(jnp.where(hcls == mx, iota, _NUM_BINS),
                  axis=0, keepdims=True)
    delta = jnp.sum(jnp.where(iota == idx, hd, 0.0), axis=0, keepdims=True)
    ang = jnp.mod(idx.astype(jnp.float32) * _ANGLE_PER_BIN + delta, _TWO_PI)

    out_ref[...] = jnp.concatenate([obj, ang, o[1:7, :]], axis=0)


@functools.partial(jax.jit, static_argnames=())
def kernel(voted_xyz, voted_features, W1, b1, W2, b2, W_obj, b_obj,
           W_box, b_box, W_hcls, b_hcls, W_hd, b_hd):
    B, N, C = voted_features.shape
    H = W1.shape[1]
    M = B * N
    T = 4096
    grid = (M // T,)

    x = voted_features.reshape(M, C)

    f32 = W1.dtype
    core = jnp.concatenate(
        [W1.reshape(-1), W2.reshape(-1), b1, b2,
         b_obj, b_box, b_hcls, b_hd,
         jnp.zeros((392 * H - 2 * H - C * H - H * H - 31,), f32)]
    ).reshape(392, H)
    heads = jnp.concatenate([W_obj, W_box, W_hcls, W_hd], axis=1)

    out = pl.pallas_call(
        _rpn_kernel,
        grid=grid,
        in_specs=[
            pl.BlockSpec((T, C), lambda i: (i, 0)),
            pl.BlockSpec((392, H), lambda i: (0, 0)),
            pl.BlockSpec((H, 31), lambda i: (0, 0)),
        ],
        out_specs=pl.BlockSpec((8, T), lambda i: (0, i)),
        out_shape=jax.ShapeDtypeStruct((8, M), jnp.float32),
    )(x, core, heads)

    obj = out[0].reshape(B, N)
    ang = out[1].reshape(B, N)
    d = out[2:8].T
    xyz = voted_xyz.reshape(M, 3)
    boxes = jnp.concatenate([xyz - d[:, 0:3], xyz + d[:, 3:6]],
                            axis=-1).reshape(B, N, 6)
    return (obj, boxes, ang)


# speedup vs baseline: 1.5323x; 1.0553x over previous
import functools

import jax
import jax.numpy as jnp
import numpy as np
from jax.experimental import pallas as pl

_NUM_BINS = 12
_ANGLE_PER_BIN = 2.0 * np.pi / _NUM_BINS
_TWO_PI = 2.0 * np.pi


def _rpn_kernel(x_ref, core_ref, heads_ref, out_ref):
    x = x_ref[...]
    w1 = core_ref[0:256, :]
    w2 = core_ref[256:384, :]
    bt = core_ref[384:392, :].T
    b1 = bt[:, 0:1]
    b2 = bt[:, 1:2]
    bh = bt[0:31, 2:3]

    h = jnp.maximum(
        jax.lax.dot_general(w1, x, (((0,), (1,)), ((), ())),
                            preferred_element_type=jnp.float32)
        + b1, 0.0)
    h = jnp.maximum(
        jax.lax.dot_general(w2, h, (((0,), (0,)), ((), ())),
                            preferred_element_type=jnp.float32)
        + b2, 0.0)
    o = (jax.lax.dot_general(heads_ref[...], h, (((0,), (0,)), ((), ())),
                             preferred_element_type=jnp.float32)
         + bh)

    obj = jax.nn.sigmoid(o[0:1, :])

    hcls = o[7:7 + _NUM_BINS, :]
    hd = o[7 + _NUM_BINS:7 + 2 * _NUM_BINS, :]
    mx = jnp.max(hcls, axis=0, keepdims=True)
    iota = jax.lax.broadcasted_iota(jnp.int32, hcls.shape, 0)
    idx = jnp.min(jnp.where(hcls == mx, iota, _NUM_BINS),
                  axis=0, keepdims=True)
    delta = jnp.sum(jnp.where(iota == idx, hd, 0.0), axis=0, keepdims=True)
    ang = jnp.mod(idx.astype(jnp.float32) * _ANGLE_PER_BIN + delta, _TWO_PI)

    d = o[1:7, :]
    signs = jnp.where(
        jax.lax.broadcasted_iota(jnp.int32, (6, 1), 0) < 3, -1.0, 1.0)
    out_ref[...] = jnp.concatenate([obj, ang, d * signs], axis=0)


@functools.partial(jax.jit, static_argnames=())
def kernel(voted_xyz, voted_features, W1, b1, W2, b2, W_obj, b_obj,
           W_box, b_box, W_hcls, b_hcls, W_hd, b_hd):
    B, N, C = voted_features.shape
    H = W1.shape[1]
    M = B * N
    T = 4096
    grid = (M // T,)

    x = voted_features.reshape(M, C)

    f32 = W1.dtype
    core = jnp.concatenate(
        [W1.reshape(-1), W2.reshape(-1), b1, b2,
         b_obj, b_box, b_hcls, b_hd,
         jnp.zeros((392 * H - 2 * H - C * H - H * H - 31,), f32)]
    ).reshape(392, H)
    heads = jnp.concatenate([W_obj, W_box, W_hcls, W_hd], axis=1)

    out = pl.pallas_call(
        _rpn_kernel,
        grid=grid,
        in_specs=[
            pl.BlockSpec((T, C), lambda i: (i, 0)),
            pl.BlockSpec((392, H), lambda i: (0, 0)),
            pl.BlockSpec((H, 31), lambda i: (0, 0)),
        ],
        out_specs=pl.BlockSpec((8, T), lambda i: (0, i)),
        out_shape=jax.ShapeDtypeStruct((8, M), jnp.float32),
    )(x, core, heads)

    obj = out[0].reshape(B, N)
    ang = out[1].reshape(B, N)
    xyz = voted_xyz.reshape(M, 3)
    xyzg = xyz[:, jnp.asarray([0, 1, 2, 0, 1, 2])]
    boxes = (xyzg + out[2:8].T).reshape(B, N, 6)
    return (obj, boxes, ang)
